# Initial kernel scaffold; baseline (speedup 1.0000x reference)
import jax
import jax.numpy as jnp
from jax.experimental import pallas as pl


def kernel(x, edge_index, W1, a_src1, a_dst1, b1, W2, a_src2, a_dst2, b2, W3, a_src3, a_dst3, b3):
    raise NotImplementedError("write your pallas kernel here")


# speedup vs baseline: 17.9221x; 17.9221x over previous
import functools

import jax
import jax.numpy as jnp
from jax import lax
from jax.experimental import pallas as pl
from jax.experimental.pallas import tpu as pltpu
from jax.experimental.pallas import tpu_sc as plsc

N = 10000
IN_FEAT = 1433
HID = 32
HEADS = 4
NUM_CLASSES = 7

NP = 10240
KP1 = 1440
CH = 128


def _matmul(x, w):
    np_, k = x.shape
    d = w.shape[1]
    bn = 256

    def body(x_ref, w_ref, o_ref):
        o_ref[...] = jnp.dot(x_ref[...], w_ref[...],
                             preferred_element_type=jnp.float32)

    return pl.pallas_call(
        body,
        grid=(np_ // bn,),
        in_specs=[pl.BlockSpec((bn, k), lambda i: (i, 0)),
                  pl.BlockSpec((k, d), lambda i: (0, 0))],
        out_specs=pl.BlockSpec((bn, d), lambda i: (i, 0)),
        out_shape=jax.ShapeDtypeStruct((np_, d), jnp.float32),
    )(x, w)


def _sc_gather(haug, alph, src, dst):
    ep = src.shape[0]
    dwh = haug.shape[1]
    info = plsc.get_sparse_core_info()
    nc, ns = info.num_cores, info.num_subcores
    nw = nc * ns
    per_w = ep // (CH * nw)
    mesh = plsc.VectorSubcoreMesh(core_axis_name="c", subcore_axis_name="s")

    @functools.partial(
        pl.kernel, mesh=mesh,
        out_type=[jax.ShapeDtypeStruct((ep, dwh), jnp.float32),
                  jax.ShapeDtypeStruct((ep, 128), jnp.float32)],
        scratch_types=[pltpu.VMEM((CH,), jnp.int32),
                       pltpu.VMEM((CH,), jnp.int32),
                       pltpu.VMEM((CH, dwh), jnp.float32),
                       pltpu.VMEM((CH, 128), jnp.float32),
                       pltpu.SemaphoreType.DMA,
                       pltpu.SemaphoreType.DMA])
    def k(haug_hbm, alph_hbm, src_hbm, dst_hbm, gs_hbm, gd_hbm,
          sidx, didx, rows, arows, s1, s2):
        wid = lax.axis_index("s") * nc + lax.axis_index("c")

        def body(i, carry):
            base = (wid * per_w + i) * CH
            pltpu.sync_copy(src_hbm.at[pl.ds(base, CH)], sidx)
            pltpu.sync_copy(dst_hbm.at[pl.ds(base, CH)], didx)
            pltpu.async_copy(haug_hbm.at[sidx], rows, s1).wait()
            pltpu.async_copy(alph_hbm.at[didx], arows, s2).wait()
            pltpu.sync_copy(rows, gs_hbm.at[pl.ds(base, CH)])
            pltpu.sync_copy(arows, gd_hbm.at[pl.ds(base, CH)])
            return carry

        lax.fori_loop(0, per_w, body, 0)

    return k(haug, alph, src, dst)


def _edge_math(gs, gd, heads, cp):
    ep, dwh = gs.shape
    hcp = heads * cp
    be = 1024

    def body(gs_ref, gd_ref, o1_ref, o2_ref):
        g = gs_ref[...]
        pieces = []
        exs = []
        for h in range(heads):
            e = g[:, hcp + h:hcp + h + 1] + gd_ref[:, 8 + h:8 + h + 1]
            e = jnp.where(e >= 0.0, e, 0.2 * e)
            ex = jnp.exp(e)
            exs.append(ex)
            pieces.append(g[:, cp * h:cp * (h + 1)] * ex)
        if hcp < 128:
            pieces.append(jnp.zeros((be, 128 - hcp), jnp.float32))
        o1_ref[...] = jnp.concatenate(pieces, axis=1)
        exs.append(jnp.zeros((be, 128 - heads), jnp.float32))
        o2_ref[...] = jnp.concatenate(exs, axis=1)

    return pl.pallas_call(
        body,
        grid=(ep // be,),
        in_specs=[pl.BlockSpec((be, dwh), lambda i: (i, 0)),
                  pl.BlockSpec((be, 128), lambda i: (i, 0))],
        out_specs=[pl.BlockSpec((be, 128), lambda i: (i, 0)),
                   pl.BlockSpec((be, 128), lambda i: (i, 0))],
        out_shape=[jax.ShapeDtypeStruct((ep, 128), jnp.float32),
                   jax.ShapeDtypeStruct((ep, 128), jnp.float32)],
    )(gs, gd)


def _sc_scatter(p, dst, zeros):
    ep, dw = p.shape
    info = plsc.get_sparse_core_info()
    nc, ns = info.num_cores, info.num_subcores
    nw = nc * ns
    per_w = ep // (CH * nw)
    mesh = plsc.VectorSubcoreMesh(core_axis_name="c", subcore_axis_name="s")

    @functools.partial(
        pl.kernel, mesh=mesh,
        out_type=jax.ShapeDtypeStruct((nc, NP, dw), jnp.float32),
        scratch_types=[pltpu.VMEM((CH,), jnp.int32),
                       pltpu.VMEM((CH, dw), jnp.float32),
                       pltpu.VMEM_SHARED((NP, dw), jnp.float32)])
    def k(p_hbm, dst_hbm, z_hbm, parts_hbm, didx, rows, accum):
        cid = lax.axis_index("c")
        sid = lax.axis_index("s")
        wid = sid * nc + cid

        @pl.when(sid == 0)
        def _():
            pltpu.sync_copy(z_hbm, accum)

        plsc.subcore_barrier()

        def body(i, carry):
            base = (wid * per_w + i) * CH
            pltpu.sync_copy(dst_hbm.at[pl.ds(base, CH)], didx)
            pltpu.sync_copy(p_hbm.at[pl.ds(base, CH)], rows)
            pltpu.sync_copy(rows, accum.at[didx], add=True)
            return carry

        lax.fori_loop(0, per_w, body, 0)
        plsc.subcore_barrier()

        @pl.when(sid == 0)
        def _():
            pltpu.sync_copy(accum, parts_hbm.at[cid])

    return k(p, dst, zeros)


def _normalize(parts1, parts2, bias2d, heads, cp, do_elu):
    ncores, np_, dw = parts1.shape
    hcp = heads * cp
    bn = 256

    def body(p1_ref, p2_ref, b_ref, o_ref):
        p1 = p1_ref[...].sum(axis=0)
        p2 = p2_ref[...].sum(axis=0)
        outs = []
        for h in range(heads):
            den = p2[:, h:h + 1] + 1e-16
            outs.append(p1[:, cp * h:cp * (h + 1)] / den)
        o = jnp.concatenate(outs, axis=1) + b_ref[...]
        if do_elu:
            o = jnp.where(o > 0.0, o, jnp.exp(o) - 1.0)
        o_ref[...] = o

    return pl.pallas_call(
        body,
        grid=(np_ // bn,),
        in_specs=[pl.BlockSpec((ncores, bn, dw), lambda i: (0, i, 0)),
                  pl.BlockSpec((ncores, bn, dw), lambda i: (0, i, 0)),
                  pl.BlockSpec((1, hcp), lambda i: (0, 0))],
        out_specs=pl.BlockSpec((bn, hcp), lambda i: (i, 0)),
        out_shape=jax.ShapeDtypeStruct((np_, hcp), jnp.float32),
    )(parts1, parts2, bias2d)


def _build_aa(a_src, a_dst, heads, cp):
    hcp = heads * cp
    c = jnp.arange(hcp)
    head = c // cp
    ch = c % cp
    nch = a_src.shape[1]
    valid = ch < nch
    chc = jnp.minimum(ch, nch - 1)
    vs = jnp.where(valid, a_src[head, chc], 0.0)
    vd = jnp.where(valid, a_dst[head, chc], 0.0)
    aa = jnp.zeros((hcp, 16), jnp.float32)
    aa = aa.at[c, head].set(vs)
    aa = aa.at[c, 8 + head].set(vd)
    return aa


def _layer(xin, w, a_src, a_dst, bias, heads, cp, src, dst, do_elu):
    hcp = heads * cp
    dwh = 128 * ((hcp + 16 + 127) // 128)
    wp = w
    if w.shape[1] != hcp:
        wp = jnp.pad(w, ((0, 0), (0, hcp - w.shape[1])))
    aa = _build_aa(a_src, a_dst, heads, cp)
    wc = jnp.concatenate([wp, wp @ aa], axis=1)
    wc = jnp.pad(wc, ((0, 0), (0, dwh - wc.shape[1])))
    haug = _matmul(xin, wc)
    alph = jnp.pad(haug[:, hcp:hcp + 16], ((0, 0), (0, 112)))
    gs, gd = _sc_gather(haug, alph, src, dst)
    p1, p2 = _edge_math(gs, gd, heads, cp)
    zeros = jnp.zeros((NP, 128), jnp.float32)
    parts1 = _sc_scatter(p1, dst, zeros)
    parts2 = _sc_scatter(p2, dst, zeros)
    bias_p = bias
    if bias.shape[0] != hcp:
        bias_p = jnp.pad(bias, (0, hcp - bias.shape[0]))
    return _normalize(parts1, parts2, bias_p.reshape(1, hcp), heads, cp,
                      do_elu)


def _impl(x, edge_index, w1, a_src1, a_dst1, b1, w2, a_src2, a_dst2, b2,
          w3, a_src3, a_dst3, b3):
    n = x.shape[0]
    e = edge_index.shape[1]
    etot = e + n
    ep = ((etot + 4095) // 4096) * 4096
    loops = jnp.arange(n, dtype=jnp.int32)
    padi = jnp.full((ep - etot,), n, jnp.int32)
    src = jnp.concatenate([edge_index[0].astype(jnp.int32), loops, padi])
    dst = jnp.concatenate([edge_index[1].astype(jnp.int32), loops, padi])

    xp = jnp.pad(x, ((0, NP - n), (0, KP1 - x.shape[1])))
    w1p = jnp.pad(w1, ((0, KP1 - w1.shape[0]), (0, 0)))

    h1 = _layer(xp, w1p, a_src1, a_dst1, b1, HEADS, HID, src, dst, True)
    h2 = _layer(h1, w2, a_src2, a_dst2, b2, HEADS, HID, src, dst, True)
    h3 = _layer(h2, w3, a_src3, a_dst3, b3, 1, 16, src, dst, False)
    return h3[:n, :NUM_CLASSES]


kernel = jax.jit(_impl)
